# Initial kernel scaffold; baseline (speedup 1.0000x reference)
import jax
import jax.numpy as jnp
from jax.experimental import pallas as pl


def kernel(x, edge_index, rbf, W1, b1, W2, b2):
    raise NotImplementedError("write your pallas kernel here")


# speedup vs baseline: 1.1231x; 1.1231x over previous
import functools

import jax
import jax.numpy as jnp
from jax import lax
from jax.experimental import pallas as pl
from jax.experimental.pallas import tpu as pltpu
from jax.experimental.pallas import tpu_sc as plsc

N = 50000
E = 800000
DIM = 64
HALF = 32

SB = 125
ROWS = E // SB
NSC = 2
NSUB = 16
R = ROWS // NSUB
CH = 16
N_PAD = 50048
NODES_PER_SUB = N_PAD // NSUB
ZROWS = 136
ZCOPIES = NODES_PER_SUB // ZROWS

BK = 3200


def _mlp_body(rbf_ref, w1_ref, b1_ref, w2_ref, b2_ref, lo_ref, hi_ref):
    dn = (((1,), (0,)), ((), ()))
    h = lax.dot_general(rbf_ref[...], w1_ref[...], dn,
                        precision=lax.Precision.HIGHEST,
                        preferred_element_type=jnp.float32) + b1_ref[...]
    bx = 0.5 * h
    act = jnp.where(bx > 14.0, h,
                    2.0 * jnp.log1p(jnp.exp(jnp.minimum(bx, 14.0))))
    h2 = lax.dot_general(act, w2_ref[...], dn,
                         precision=lax.Precision.HIGHEST,
                         preferred_element_type=jnp.float32) + b2_ref[...]
    lo_ref[...] = h2[:, :HALF]
    hi_ref[...] = h2[:, HALF:]


def _edge_mlp(rbf, W1, b1, W2, b2):
    grid = (E // BK,)
    return pl.pallas_call(
        _mlp_body,
        grid=grid,
        in_specs=[
            pl.BlockSpec((BK, DIM), lambda i: (i, 0)),
            pl.BlockSpec((DIM, DIM), lambda i: (0, 0)),
            pl.BlockSpec((1, DIM), lambda i: (0, 0)),
            pl.BlockSpec((DIM, DIM), lambda i: (0, 0)),
            pl.BlockSpec((1, DIM), lambda i: (0, 0)),
        ],
        out_specs=[
            pl.BlockSpec((BK, HALF), lambda i: (i, 0)),
            pl.BlockSpec((BK, HALF), lambda i: (i, 0)),
        ],
        out_shape=[
            jax.ShapeDtypeStruct((E, HALF), jnp.float32),
            jax.ShapeDtypeStruct((E, HALF), jnp.float32),
        ],
    )(rbf, W1, b1.reshape(1, DIM), W2, b2.reshape(1, DIM))


def _sc_body(x_hbm, src_hbm, dst_hbm, hlo_hbm, hhi_hbm, out_hbm,
             acc, idx_s, idx_d, xr, hv, msg, zbuf, sem):
    cid = lax.axis_index("c")
    sid = lax.axis_index("s")

    zeros16 = jnp.zeros((16,), jnp.float32)

    @pl.loop(0, ZROWS)
    def _(k):
        zbuf[k, pl.ds(0, 16)] = zeros16
        zbuf[k, pl.ds(16, 16)] = zeros16

    acc_base = sid * NODES_PER_SUB

    @pl.loop(0, ZCOPIES)
    def _(i):
        pltpu.sync_copy(zbuf, acc.at[pl.ds(acc_base + i * ZROWS, ZROWS)])

    plsc.subcore_barrier()

    @pl.loop(0, R // CH)
    def _(ci):
        row0 = sid * R + ci * CH
        pltpu.sync_copy(src_hbm.at[pl.ds(row0, CH)], idx_s)
        pltpu.sync_copy(dst_hbm.at[pl.ds(row0, CH)], idx_d)

        @pl.loop(0, CH)
        def _(j):
            row = row0 + j
            pltpu.async_copy(x_hbm.at[idx_s.at[j]], xr, sem).wait()

            @pl.when(cid == 0)
            def _():
                pltpu.sync_copy(hlo_hbm.at[row], hv)

            @pl.when(cid == 1)
            def _():
                pltpu.sync_copy(hhi_hbm.at[row], hv)

            @pl.when(cid == 0)
            def _():
                @pl.loop(0, SB, unroll=4)
                def _(k):
                    msg[k, pl.ds(0, 16)] = xr[k, pl.ds(0, 16)] * hv[k, pl.ds(0, 16)]
                    msg[k, pl.ds(16, 16)] = xr[k, pl.ds(16, 16)] * hv[k, pl.ds(16, 16)]

            @pl.when(cid == 1)
            def _():
                @pl.loop(0, SB, unroll=4)
                def _(k):
                    msg[k, pl.ds(0, 16)] = xr[k, pl.ds(32, 16)] * hv[k, pl.ds(0, 16)]
                    msg[k, pl.ds(16, 16)] = xr[k, pl.ds(48, 16)] * hv[k, pl.ds(16, 16)]

            pltpu.sync_copy(msg, acc.at[idx_d.at[j]], add=True)

    plsc.subcore_barrier()

    pltpu.sync_copy(acc.at[pl.ds(acc_base, NODES_PER_SUB)],
                    out_hbm.at[cid, pl.ds(acc_base, NODES_PER_SUB)])


def _sc_aggregate(x, src3, dst3, hlo3, hhi3):
    mesh = plsc.VectorSubcoreMesh(core_axis_name="c", subcore_axis_name="s")
    f = pl.kernel(
        _sc_body,
        out_type=jax.ShapeDtypeStruct((NSC, N_PAD, HALF), jnp.float32),
        mesh=mesh,
        compiler_params=pltpu.CompilerParams(use_tc_tiling_on_sc=False),
        scratch_types=[
            pltpu.VMEM_SHARED((N_PAD, HALF), jnp.float32),
            pltpu.VMEM((CH, SB), jnp.int32),
            pltpu.VMEM((CH, SB), jnp.int32),
            pltpu.VMEM((SB, DIM), jnp.float32),
            pltpu.VMEM((SB, HALF), jnp.float32),
            pltpu.VMEM((SB, HALF), jnp.float32),
            pltpu.VMEM((ZROWS, HALF), jnp.float32),
            pltpu.SemaphoreType.DMA,
        ],
    )
    return f(x, src3, dst3, hlo3, hhi3)


def kernel(x, edge_index, rbf, W1, b1, W2, b2):
    hlo, hhi = _edge_mlp(rbf, W1, b1, W2, b2)
    src3 = edge_index[0].reshape(ROWS, SB)
    dst3 = edge_index[1].reshape(ROWS, SB)
    hlo3 = hlo.reshape(ROWS, SB, HALF)
    hhi3 = hhi.reshape(ROWS, SB, HALF)
    out = _sc_aggregate(x, src3, dst3, hlo3, hhi3)
    return jnp.concatenate([out[0, :N], out[1, :N]], axis=1)


# speedup vs baseline: 1.5073x; 1.3421x over previous
import functools

import jax
import jax.numpy as jnp
from jax import lax
from jax.experimental import pallas as pl
from jax.experimental.pallas import tpu as pltpu
from jax.experimental.pallas import tpu_sc as plsc

N = 50000
E = 800000
DIM = 64
HALF = 32

SB = 125
ROWS = E // SB
NSC = 2
NSUB = 16
R = ROWS // NSUB
CH = 16
N_PAD = 50048
NODES_PER_SUB = N_PAD // NSUB
ZROWS = 136
ZCOPIES = NODES_PER_SUB // ZROWS

BK = 3200


def _mlp_body(rbf_ref, w1_ref, b1_ref, w2_ref, b2_ref, lo_ref, hi_ref):
    dn = (((1,), (0,)), ((), ()))
    h = lax.dot_general(rbf_ref[...], w1_ref[...], dn,
                        preferred_element_type=jnp.float32) + b1_ref[...]
    bx = 0.5 * h
    act = jnp.where(bx > 14.0, h,
                    2.0 * jnp.log1p(jnp.exp(jnp.minimum(bx, 14.0))))
    h2 = lax.dot_general(act, w2_ref[...], dn,
                         preferred_element_type=jnp.float32) + b2_ref[...]
    lo_ref[...] = h2[:, :HALF]
    hi_ref[...] = h2[:, HALF:]


def _edge_mlp(rbf, W1, b1, W2, b2):
    grid = (E // BK,)
    return pl.pallas_call(
        _mlp_body,
        grid=grid,
        in_specs=[
            pl.BlockSpec((BK, DIM), lambda i: (i, 0)),
            pl.BlockSpec((DIM, DIM), lambda i: (0, 0)),
            pl.BlockSpec((1, DIM), lambda i: (0, 0)),
            pl.BlockSpec((DIM, DIM), lambda i: (0, 0)),
            pl.BlockSpec((1, DIM), lambda i: (0, 0)),
        ],
        out_specs=[
            pl.BlockSpec((BK, HALF), lambda i: (i, 0)),
            pl.BlockSpec((BK, HALF), lambda i: (i, 0)),
        ],
        out_shape=[
            jax.ShapeDtypeStruct((E, HALF), jnp.float32),
            jax.ShapeDtypeStruct((E, HALF), jnp.float32),
        ],
    )(rbf, W1, b1.reshape(1, DIM), W2, b2.reshape(1, DIM))


def _sc_body(x_hbm, src_hbm, dst_hbm, hlo_hbm, hhi_hbm, out_hbm,
             acc, idx_s, idx_d, xr, hv, msg, zbuf, sem):
    cid = lax.axis_index("c")
    sid = lax.axis_index("s")

    zeros16 = jnp.zeros((16,), jnp.float32)

    @pl.loop(0, ZROWS)
    def _(k):
        zbuf[k, pl.ds(0, 16)] = zeros16
        zbuf[k, pl.ds(16, 16)] = zeros16

    acc_base = sid * NODES_PER_SUB

    @pl.loop(0, ZCOPIES)
    def _(i):
        pltpu.sync_copy(zbuf, acc.at[pl.ds(acc_base + i * ZROWS, ZROWS)])

    plsc.subcore_barrier()

    @pl.loop(0, R // CH)
    def _(ci):
        row0 = sid * R + ci * CH
        pltpu.sync_copy(src_hbm.at[pl.ds(row0, CH)], idx_s)
        pltpu.sync_copy(dst_hbm.at[pl.ds(row0, CH)], idx_d)

        @pl.loop(0, CH)
        def _(j):
            row = row0 + j
            pltpu.async_copy(x_hbm.at[idx_s.at[j]], xr, sem).wait()

            @pl.when(cid == 0)
            def _():
                pltpu.sync_copy(hlo_hbm.at[row], hv)

            @pl.when(cid == 1)
            def _():
                pltpu.sync_copy(hhi_hbm.at[row], hv)

            @pl.when(cid == 0)
            def _():
                @pl.loop(0, SB, unroll=4)
                def _(k):
                    msg[k, pl.ds(0, 16)] = xr[k, pl.ds(0, 16)] * hv[k, pl.ds(0, 16)]
                    msg[k, pl.ds(16, 16)] = xr[k, pl.ds(16, 16)] * hv[k, pl.ds(16, 16)]

            @pl.when(cid == 1)
            def _():
                @pl.loop(0, SB, unroll=4)
                def _(k):
                    msg[k, pl.ds(0, 16)] = xr[k, pl.ds(32, 16)] * hv[k, pl.ds(0, 16)]
                    msg[k, pl.ds(16, 16)] = xr[k, pl.ds(48, 16)] * hv[k, pl.ds(16, 16)]

            pltpu.sync_copy(msg, acc.at[idx_d.at[j]], add=True)

    plsc.subcore_barrier()

    pltpu.sync_copy(acc.at[pl.ds(acc_base, NODES_PER_SUB)],
                    out_hbm.at[cid, pl.ds(acc_base, NODES_PER_SUB)])


def _sc_aggregate(x, src3, dst3, hlo3, hhi3):
    mesh = plsc.VectorSubcoreMesh(core_axis_name="c", subcore_axis_name="s")
    f = pl.kernel(
        _sc_body,
        out_type=jax.ShapeDtypeStruct((NSC, N_PAD, HALF), jnp.float32),
        mesh=mesh,
        compiler_params=pltpu.CompilerParams(use_tc_tiling_on_sc=False),
        scratch_types=[
            pltpu.VMEM_SHARED((N_PAD, HALF), jnp.float32),
            pltpu.VMEM((CH, SB), jnp.int32),
            pltpu.VMEM((CH, SB), jnp.int32),
            pltpu.VMEM((SB, DIM), jnp.float32),
            pltpu.VMEM((SB, HALF), jnp.float32),
            pltpu.VMEM((SB, HALF), jnp.float32),
            pltpu.VMEM((ZROWS, HALF), jnp.float32),
            pltpu.SemaphoreType.DMA,
        ],
    )
    return f(x, src3, dst3, hlo3, hhi3)


def kernel(x, edge_index, rbf, W1, b1, W2, b2):
    hlo, hhi = _edge_mlp(rbf, W1, b1, W2, b2)
    src3 = edge_index[0].reshape(ROWS, SB)
    dst3 = edge_index[1].reshape(ROWS, SB)
    hlo3 = hlo.reshape(ROWS, SB, HALF)
    hhi3 = hhi.reshape(ROWS, SB, HALF)
    out = _sc_aggregate(x, src3, dst3, hlo3, hhi3)
    return jnp.concatenate([out[0, :N], out[1, :N]], axis=1)
